# baseline (speedup 1.0000x reference)
import functools

import jax
import jax.numpy as jnp
from jax import lax
from jax.experimental import pallas as pl
from jax.experimental.pallas import tpu as pltpu
from jax.experimental.pallas import tpu_sc as plsc

NC = 2
NS = 16
L = 16
NW = NC * NS

B = 16384
D = 128
BPW = B // NW
CHUNK = 128
NCHUNK = BPW // CHUNK
GROUPS = CHUNK // L
KD = D // L
SPLIT = 2 * L

_mesh = plsc.VectorSubcoreMesh(
    core_axis_name="c", subcore_axis_name="s", num_cores=NC, num_subcores=NS
)


@functools.partial(
    pl.kernel,
    out_type=jax.ShapeDtypeStruct((B,), jnp.float32),
    mesh=_mesh,
    compiler_params=pltpu.CompilerParams(needs_layout_passes=False),
    scratch_types=[
        pltpu.VMEM((BPW,), jnp.int32),
        pltpu.VMEM((BPW,), jnp.int32),
        pltpu.VMEM((CHUNK, D), jnp.float32),
        pltpu.VMEM((CHUNK, D), jnp.float32),
        pltpu.VMEM((CHUNK, D), jnp.float32),
        pltpu.VMEM((CHUNK, D), jnp.float32),
        pltpu.VMEM((1, D), jnp.float32),
        pltpu.VMEM((1,), jnp.float32),
        pltpu.VMEM((BPW,), jnp.float32),
        pltpu.VMEM((CHUNK, L), jnp.float32),
        pltpu.VMEM((CHUNK, L), jnp.float32),
        pltpu.SemaphoreType.DMA,
        pltpu.SemaphoreType.DMA,
        pltpu.SemaphoreType.DMA,
        pltpu.SemaphoreType.DMA,
        pltpu.SemaphoreType.DMA,
        pltpu.SemaphoreType.DMA,
    ],
)
def _mf_kernel(uid_hbm, mid_hbm, ut_hbm, mt_hbm, w_hbm, b_hbm, out_hbm,
               uidx_v, midx_v, urows0, urows1, mrows0, mrows1,
               w_v, b_v, out_v, accA, accB,
               semu0, semu1, semm0, semm1, semi, semw):
    wid = lax.axis_index("s") * NC + lax.axis_index("c")
    base = wid * BPW

    ci = pltpu.async_copy(uid_hbm.at[pl.ds(base, BPW)], uidx_v, semi)
    cj = pltpu.async_copy(mid_hbm.at[pl.ds(base, BPW)], midx_v, semi)
    cw = pltpu.async_copy(w_hbm, w_v, semw)
    cb = pltpu.async_copy(b_hbm, b_v, semw)
    ci.wait()
    cj.wait()
    cw.wait()
    cb.wait()

    urows = (urows0, urows1)
    mrows = (mrows0, mrows1)
    semu = (semu0, semu1)
    semm = (semm0, semm1)
    accs = (accA, accB)

    def start(c):
        buf = c % 2
        cu = pltpu.async_copy(
            ut_hbm.at[uidx_v.at[pl.ds(c * CHUNK, CHUNK)]], urows[buf], semu[buf])
        cm = pltpu.async_copy(
            mt_hbm.at[midx_v.at[pl.ds(c * CHUNK, CHUNK)]], mrows[buf], semm[buf])
        return cu, cm

    cu0, cm0 = start(0)
    cu1, cm1 = start(1)

    wk = [w_v[0, pl.ds(k * L, L)] for k in range(KD)]
    lane = lax.broadcasted_iota(jnp.int32, (L,), 0)
    zero = jnp.zeros((L,), jnp.int32)
    bvec = plsc.load_gather(b_v, [zero])

    def make_body(c):
        u_v, m_v = urows[c % 2], mrows[c % 2]
        acc_v = accs[c % 2]
        out_base = c * CHUNK

        def body(g, gcarry):
            row0 = g * L

            def load_row(r):
                return ([u_v[r, pl.ds(k * L, L)] for k in range(KD)],
                        [m_v[r, pl.ds(k * L, L)] for k in range(KD)])

            us, ms = load_row(row0)
            for j in range(L):
                cu_, cm_ = us, ms
                if j + 1 < L:
                    us, ms = load_row(row0 + j + 1)
                ps = [cu_[k] * cm_[k] * wk[k] for k in range(KD)]
                while len(ps) > 1:
                    ps = [ps[i] + ps[i + 1] for i in range(0, len(ps), 2)]
                acc_v[j, :] = ps[0]
            cols = [plsc.load_gather(acc_v, [lane, jnp.full((L,), l, jnp.int32)])
                    for l in range(L)]
            while len(cols) > 1:
                cols = [cols[i] + cols[i + 1] for i in range(0, len(cols), 2)]
            out_v[pl.ds(out_base + row0, L)] = cols[0] + bvec
            return gcarry

        return body

    outc = []

    def flush(c):
        outc.append(pltpu.async_copy(
            out_v.at[pl.ds(c * CHUNK, CHUNK)],
            out_hbm.at[pl.ds(base + c * CHUNK, CHUNK)], semw))

    cu0.wait()
    cm0.wait()
    lax.fori_loop(0, GROUPS, make_body(0), 0)
    cu2, cm2 = start(2)
    flush(0)

    cu1.wait()
    cm1.wait()
    lax.fori_loop(0, GROUPS, make_body(1), 0)
    cu3, cm3 = start(3)
    flush(1)

    cu2.wait()
    cm2.wait()
    lax.fori_loop(0, GROUPS, make_body(2), 0)
    flush(2)

    cu3.wait()
    cm3.wait()
    lax.fori_loop(0, GROUPS, make_body(3), 0)
    flush(3)

    for cpy in outc:
        cpy.wait()


def kernel(user_ids, movie_ids, user_table, movie_table, fc_w, fc_b):
    uid = user_ids if user_ids.dtype == jnp.int32 else user_ids.astype(jnp.int32)
    mid = movie_ids if movie_ids.dtype == jnp.int32 else movie_ids.astype(jnp.int32)
    return _mf_kernel(uid, mid, user_table, movie_table, fc_w, fc_b)


# speedup vs baseline: 1.0314x; 1.0314x over previous
import functools

import jax
import jax.numpy as jnp
from jax import lax
from jax.experimental import pallas as pl
from jax.experimental.pallas import tpu as pltpu
from jax.experimental.pallas import tpu_sc as plsc

NC = 2
NS = 16
L = 16
NW = NC * NS

B = 16384
D = 128
BPW = B // NW
CHUNK = 128
NCHUNK = BPW // CHUNK
GROUPS = CHUNK // L
KD = D // L
SPLIT = 2 * L

_mesh = plsc.VectorSubcoreMesh(
    core_axis_name="c", subcore_axis_name="s", num_cores=NC, num_subcores=NS
)


@functools.partial(
    pl.kernel,
    out_type=jax.ShapeDtypeStruct((B,), jnp.float32),
    mesh=_mesh,
    compiler_params=pltpu.CompilerParams(needs_layout_passes=False),
    scratch_types=[
        pltpu.VMEM((BPW,), jnp.int32),
        pltpu.VMEM((BPW,), jnp.int32),
        pltpu.VMEM((CHUNK, D), jnp.float32),
        pltpu.VMEM((CHUNK, D), jnp.float32),
        pltpu.VMEM((CHUNK, D), jnp.float32),
        pltpu.VMEM((CHUNK, D), jnp.float32),
        pltpu.VMEM((1, D), jnp.float32),
        pltpu.VMEM((1,), jnp.float32),
        pltpu.VMEM((BPW,), jnp.float32),
        pltpu.VMEM((L, L), jnp.float32),
        pltpu.SemaphoreType.DMA,
        pltpu.SemaphoreType.DMA,
        pltpu.SemaphoreType.DMA,
        pltpu.SemaphoreType.DMA,
        pltpu.SemaphoreType.DMA,
        pltpu.SemaphoreType.DMA,
    ],
)
def _mf_kernel(uid_hbm, mid_hbm, ut_hbm, mt_hbm, w_hbm, b_hbm, out_hbm,
               uidx_v, midx_v, urows0, urows1, mrows0, mrows1,
               w_v, b_v, out_v, acc_v,
               semu0, semu1, semm0, semm1, semi, semw):
    wid = lax.axis_index("s") * NC + lax.axis_index("c")
    base = wid * BPW

    ci = pltpu.async_copy(uid_hbm.at[pl.ds(base, BPW)], uidx_v, semi)
    cj = pltpu.async_copy(mid_hbm.at[pl.ds(base, BPW)], midx_v, semi)
    cw = pltpu.async_copy(w_hbm, w_v, semw)
    cb = pltpu.async_copy(b_hbm, b_v, semw)
    ci.wait()
    cj.wait()

    urows = (urows0, urows1)
    mrows = (mrows0, mrows1)
    semu = (semu0, semu1)
    semm = (semm0, semm1)

    def start(c):
        buf = c % 2
        cu = pltpu.async_copy(
            ut_hbm.at[uidx_v.at[pl.ds(c * CHUNK, CHUNK)]], urows[buf], semu[buf])
        cm = pltpu.async_copy(
            mt_hbm.at[midx_v.at[pl.ds(c * CHUNK, CHUNK)]], mrows[buf], semm[buf])
        return cu, cm

    cu0, cm0 = start(0)
    cu1, cm1 = start(1)
    cw.wait()
    cb.wait()

    wk = [w_v[0, pl.ds(k * L, L)] for k in range(KD)]
    lane = lax.broadcasted_iota(jnp.int32, (L,), 0)
    zero = jnp.zeros((L,), jnp.int32)
    bvec = plsc.load_gather(b_v, [zero])

    def make_body(c):
        u_v, m_v = urows[c % 2], mrows[c % 2]
        out_base = c * CHUNK

        def body(g, gcarry):
            row0 = g * L

            def load_row(r):
                return ([u_v[r, pl.ds(k * L, L)] for k in range(KD)],
                        [m_v[r, pl.ds(k * L, L)] for k in range(KD)])

            us, ms = load_row(row0)
            for j in range(L):
                cu_, cm_ = us, ms
                if j + 1 < L:
                    us, ms = load_row(row0 + j + 1)
                ps = [cu_[k] * cm_[k] * wk[k] for k in range(KD)]
                while len(ps) > 1:
                    ps = [ps[i] + ps[i + 1] for i in range(0, len(ps), 2)]
                acc_v[j, :] = ps[0]
            cols = [plsc.load_gather(acc_v, [lane, jnp.full((L,), l, jnp.int32)])
                    for l in range(L)]
            while len(cols) > 1:
                cols = [cols[i] + cols[i + 1] for i in range(0, len(cols), 2)]
            out_v[pl.ds(out_base + row0, L)] = cols[0] + bvec
            return gcarry

        return body

    cu0.wait()
    cm0.wait()
    lax.fori_loop(0, GROUPS, make_body(0), 0)
    cu2, cm2 = start(2)

    cu1.wait()
    cm1.wait()
    lax.fori_loop(0, GROUPS, make_body(1), 0)
    cu3, cm3 = start(3)

    cu2.wait()
    cm2.wait()
    lax.fori_loop(0, GROUPS, make_body(2), 0)

    cu3.wait()
    cm3.wait()
    lax.fori_loop(0, GROUPS, make_body(3), 0)

    pltpu.sync_copy(out_v, out_hbm.at[pl.ds(base, BPW)])


def kernel(user_ids, movie_ids, user_table, movie_table, fc_w, fc_b):
    uid = user_ids if user_ids.dtype == jnp.int32 else user_ids.astype(jnp.int32)
    mid = movie_ids if movie_ids.dtype == jnp.int32 else movie_ids.astype(jnp.int32)
    return _mf_kernel(uid, mid, user_table, movie_table, fc_w, fc_b)
